# Initial kernel scaffold; baseline (speedup 1.0000x reference)
import jax
import jax.numpy as jnp
from jax.experimental import pallas as pl


def kernel(x, edge_index, edge_attr, W_in, b_in, Wm1, bm1, Wm2, bm2, Wu1, bu1, Wu2, bu2, W_pred, b_pred):
    raise NotImplementedError("write your pallas kernel here")


# speedup vs baseline: 1.8706x; 1.8706x over previous
import functools

import jax
import jax.numpy as jnp
from jax import lax
from jax.experimental import pallas as pl
from jax.experimental.pallas import tpu as pltpu
from jax.experimental.pallas import tpu_sc as plsc

NC = 2
NS = 16
LANES = 16
CH = 80

F32 = jnp.float32


def _inproj(x, w, b, bn=1000):
    n, din = x.shape
    d = w.shape[1]

    def body(x_ref, w_ref, b_ref, o_ref):
        o_ref[...] = (
            jnp.dot(x_ref[...], w_ref[...], preferred_element_type=F32)
            + b_ref[...]
        )

    return pl.pallas_call(
        body,
        grid=(n // bn,),
        in_specs=[
            pl.BlockSpec((bn, din), lambda i: (i, 0)),
            pl.BlockSpec((din, d), lambda i: (0, 0)),
            pl.BlockSpec((1, d), lambda i: (0, 0)),
        ],
        out_specs=pl.BlockSpec((bn, d), lambda i: (i, 0)),
        out_shape=jax.ShapeDtypeStruct((n, d), F32),
    )(x, w, b)


def _ab_proj(h, wa, wb, bn=1000):
    n, d = h.shape

    def body(h_ref, wa_ref, wb_ref, a_ref, b_ref):
        hh = h_ref[...]
        a_ref[...] = jnp.dot(hh, wa_ref[...], preferred_element_type=F32)
        b_ref[...] = jnp.dot(hh, wb_ref[...], preferred_element_type=F32)

    return pl.pallas_call(
        body,
        grid=(n // bn,),
        in_specs=[
            pl.BlockSpec((bn, d), lambda i: (i, 0)),
            pl.BlockSpec((d, d), lambda i: (0, 0)),
            pl.BlockSpec((d, d), lambda i: (0, 0)),
        ],
        out_specs=[
            pl.BlockSpec((bn, d), lambda i: (i, 0)),
            pl.BlockSpec((bn, d), lambda i: (i, 0)),
        ],
        out_shape=[
            jax.ShapeDtypeStruct((n, d), F32),
            jax.ShapeDtypeStruct((n, d), F32),
        ],
    )(h, wa, wb)


def _edge_mlp(g, ea, we, b1, w2, b2, eb=1280):
    e, d = g.shape
    de = ea.shape[1]

    def body(g_ref, ea_ref, we_ref, b1_ref, w2_ref, b2_ref, o_ref):
        t = (
            g_ref[...]
            + jnp.dot(ea_ref[...], we_ref[...], preferred_element_type=F32)
            + b1_ref[...]
        )
        u = jnp.maximum(t, 0.0)
        v = jnp.dot(u, w2_ref[...], preferred_element_type=F32) + b2_ref[...]
        o_ref[...] = jnp.maximum(v, 0.0)

    return pl.pallas_call(
        body,
        grid=(e // eb,),
        in_specs=[
            pl.BlockSpec((eb, d), lambda i: (i, 0)),
            pl.BlockSpec((eb, de), lambda i: (i, 0)),
            pl.BlockSpec((de, d), lambda i: (0, 0)),
            pl.BlockSpec((1, d), lambda i: (0, 0)),
            pl.BlockSpec((d, d), lambda i: (0, 0)),
            pl.BlockSpec((1, d), lambda i: (0, 0)),
        ],
        out_specs=pl.BlockSpec((eb, d), lambda i: (i, 0)),
        out_shape=jax.ShapeDtypeStruct((e, d), F32),
    )(g, ea, we, b1, w2, b2)


def _update_mlp(h, aggr, wa, wb, b1, w2, b2, bn=1000):
    n, d = h.shape

    def body(h_ref, ag_ref, wa_ref, wb_ref, b1_ref, w2_ref, b2_ref, o_ref):
        hh = h_ref[...]
        z = (
            jnp.dot(hh, wa_ref[...], preferred_element_type=F32)
            + jnp.dot(ag_ref[...], wb_ref[...], preferred_element_type=F32)
            + b1_ref[...]
        )
        z = jnp.maximum(z, 0.0)
        u = jnp.dot(z, w2_ref[...], preferred_element_type=F32) + b2_ref[...]
        o_ref[...] = hh + jnp.maximum(u, 0.0)

    return pl.pallas_call(
        body,
        grid=(n // bn,),
        in_specs=[
            pl.BlockSpec((bn, d), lambda i: (i, 0)),
            pl.BlockSpec((bn, d), lambda i: (i, 0)),
            pl.BlockSpec((d, d), lambda i: (0, 0)),
            pl.BlockSpec((d, d), lambda i: (0, 0)),
            pl.BlockSpec((1, d), lambda i: (0, 0)),
            pl.BlockSpec((d, d), lambda i: (0, 0)),
            pl.BlockSpec((1, d), lambda i: (0, 0)),
        ],
        out_specs=pl.BlockSpec((bn, d), lambda i: (i, 0)),
        out_shape=jax.ShapeDtypeStruct((n, d), F32),
    )(h, aggr, wa, wb, b1, w2, b2)


def _pred_head(h, w, b, bn=1000):
    n, d = h.shape

    def body(h_ref, w_ref, b_ref, o_ref):
        o_ref[...] = (
            jnp.dot(h_ref[...], w_ref[...], preferred_element_type=F32)
            + b_ref[...]
        )

    return pl.pallas_call(
        body,
        grid=(n // bn,),
        in_specs=[
            pl.BlockSpec((bn, d), lambda i: (i, 0)),
            pl.BlockSpec((d, 1), lambda i: (0, 0)),
            pl.BlockSpec((1, 1), lambda i: (0, 0)),
        ],
        out_specs=pl.BlockSpec((bn, 1), lambda i: (i, 0)),
        out_shape=jax.ShapeDtypeStruct((n, 1), F32),
    )(h, w, b)


@functools.cache
def _make_gather_add(n, e, d):
    ept = e // (NC * NS)
    nch = ept // CH
    mesh = plsc.VectorSubcoreMesh(core_axis_name="c", subcore_axis_name="s")

    @functools.partial(
        pl.kernel,
        out_type=jax.ShapeDtypeStruct((e, d), F32),
        mesh=mesh,
        scratch_types=[
            pltpu.VMEM((CH,), jnp.int32),
            pltpu.VMEM((CH,), jnp.int32),
            pltpu.VMEM((CH, d), F32),
            pltpu.VMEM((CH, d), F32),
            pltpu.SemaphoreType.DMA,
            pltpu.SemaphoreType.DMA,
        ],
    )
    def gather_add(a_hbm, b_hbm, dst_hbm, src_hbm, g_hbm,
                   dst_v, src_v, a_v, b_v, sem_a, sem_b):
        wid = lax.axis_index("s") * NC + lax.axis_index("c")
        base = wid * ept

        def chunk(k, carry):
            e0 = base + k * CH
            pltpu.sync_copy(dst_hbm.at[pl.ds(e0, CH)], dst_v)
            pltpu.sync_copy(src_hbm.at[pl.ds(e0, CH)], src_v)
            ca = pltpu.async_copy(a_hbm.at[dst_v], a_v, sem_a)
            cb = pltpu.async_copy(b_hbm.at[src_v], b_v, sem_b)
            ca.wait()
            cb.wait()

            def row(r, c2):
                for j in range(d // LANES):
                    sl = pl.ds(j * LANES, LANES)
                    a_v[r, sl] = a_v[r, sl] + b_v[r, sl]
                return c2

            lax.fori_loop(0, CH, row, 0)
            pltpu.sync_copy(a_v, g_hbm.at[pl.ds(e0, CH)])
            return carry

        lax.fori_loop(0, nch, chunk, 0)

    return gather_add


@functools.cache
def _make_scatter_add(n_pad, e, d):
    dh = d // NC
    eps = e // NS
    nch = eps // CH
    nps = n_pad // NS
    mesh = plsc.VectorSubcoreMesh(core_axis_name="c", subcore_axis_name="s")

    @functools.partial(
        pl.kernel,
        out_type=jax.ShapeDtypeStruct((n_pad, d), F32),
        mesh=mesh,
        scratch_types=[
            pltpu.VMEM_SHARED((n_pad, dh), F32),
            pltpu.VMEM((CH, dh), F32),
            pltpu.VMEM((CH,), jnp.int32),
        ],
    )
    def scatter_add(v_hbm, dst_hbm, z_hbm, aggr_hbm, acc_s, v_v, idx_v):
        cid = lax.axis_index("c")
        sid = lax.axis_index("s")
        r0 = sid * nps
        c0 = cid * dh
        pltpu.sync_copy(z_hbm.at[pl.ds(r0, nps)], acc_s.at[pl.ds(r0, nps)])
        plsc.subcore_barrier()
        base = sid * eps

        def chunk(k, carry):
            e0 = base + k * CH
            pltpu.sync_copy(dst_hbm.at[pl.ds(e0, CH)], idx_v)
            pltpu.sync_copy(v_hbm.at[pl.ds(e0, CH), pl.ds(c0, dh)], v_v)
            pltpu.sync_copy(v_v, acc_s.at[idx_v], add=True)
            return carry

        lax.fori_loop(0, nch, chunk, 0)
        plsc.subcore_barrier()
        pltpu.sync_copy(acc_s.at[pl.ds(r0, nps)],
                        aggr_hbm.at[pl.ds(r0, nps), pl.ds(c0, dh)])

    return scatter_add


def kernel(x, edge_index, edge_attr, W_in, b_in, Wm1, bm1, Wm2, bm2,
           Wu1, bu1, Wu2, bu2, W_pred, b_pred):
    n, _ = x.shape
    e = edge_index.shape[1]
    d = W_in.shape[1]
    nl = Wm1.shape[0]

    src = edge_index[0]
    dst = edge_index[1]
    n_pad = ((n + NS * 8 - 1) // (NS * 8)) * NS * 8
    zeros_half = jnp.zeros((n_pad, d // NC), F32)

    gather_add = _make_gather_add(n, e, d)
    scatter_add = _make_scatter_add(n_pad, e, d)

    h = _inproj(x, W_in, b_in.reshape(1, -1))
    for l in range(nl):
        a, b = _ab_proj(h, Wm1[l, :d], Wm1[l, d:2 * d])
        g = gather_add(a, b, dst, src)
        v = _edge_mlp(g, edge_attr, Wm1[l, 2 * d:], bm1[l].reshape(1, -1),
                      Wm2[l], bm2[l].reshape(1, -1))
        aggr = scatter_add(v, dst, zeros_half)[:n]
        h = _update_mlp(h, aggr, Wu1[l, :d], Wu1[l, d:], bu1[l].reshape(1, -1),
                        Wu2[l], bu2[l].reshape(1, -1))
    return _pred_head(h, W_pred, b_pred.reshape(1, -1))


# speedup vs baseline: 2.0612x; 1.1019x over previous
import functools

import jax
import jax.numpy as jnp
from jax import lax
from jax.experimental import pallas as pl
from jax.experimental.pallas import tpu as pltpu
from jax.experimental.pallas import tpu_sc as plsc

NC = 2
NS = 16
LANES = 16
CH = 40

F32 = jnp.float32


def _inproj(x, w, b, bn=1000):
    n, din = x.shape
    d = w.shape[1]

    def body(x_ref, w_ref, b_ref, o_ref):
        o_ref[...] = (
            jnp.dot(x_ref[...], w_ref[...], preferred_element_type=F32)
            + b_ref[...]
        )

    return pl.pallas_call(
        body,
        grid=(n // bn,),
        in_specs=[
            pl.BlockSpec((bn, din), lambda i: (i, 0)),
            pl.BlockSpec((din, d), lambda i: (0, 0)),
            pl.BlockSpec((1, d), lambda i: (0, 0)),
        ],
        out_specs=pl.BlockSpec((bn, d), lambda i: (i, 0)),
        out_shape=jax.ShapeDtypeStruct((n, d), F32),
    )(x, w, b)


def _ab_proj(h, wa, wb, bn=1000):
    n, d = h.shape

    def body(h_ref, wa_ref, wb_ref, a_ref, b_ref):
        hh = h_ref[...]
        a_ref[...] = jnp.dot(hh, wa_ref[...], preferred_element_type=F32)
        b_ref[...] = jnp.dot(hh, wb_ref[...], preferred_element_type=F32)

    return pl.pallas_call(
        body,
        grid=(n // bn,),
        in_specs=[
            pl.BlockSpec((bn, d), lambda i: (i, 0)),
            pl.BlockSpec((d, d), lambda i: (0, 0)),
            pl.BlockSpec((d, d), lambda i: (0, 0)),
        ],
        out_specs=[
            pl.BlockSpec((bn, d), lambda i: (i, 0)),
            pl.BlockSpec((bn, d), lambda i: (i, 0)),
        ],
        out_shape=[
            jax.ShapeDtypeStruct((n, d), F32),
            jax.ShapeDtypeStruct((n, d), F32),
        ],
    )(h, wa, wb)


def _edge_mlp(g, ea, we, b1, w2, b2, eb=1280):
    e, d = g.shape
    de = ea.shape[1]

    def body(g_ref, ea_ref, we_ref, b1_ref, w2_ref, b2_ref, o_ref):
        t = (
            g_ref[...]
            + jnp.dot(ea_ref[...], we_ref[...], preferred_element_type=F32)
            + b1_ref[...]
        )
        u = jnp.maximum(t, 0.0)
        v = jnp.dot(u, w2_ref[...], preferred_element_type=F32) + b2_ref[...]
        o_ref[...] = jnp.maximum(v, 0.0)

    return pl.pallas_call(
        body,
        grid=(e // eb,),
        in_specs=[
            pl.BlockSpec((eb, d), lambda i: (i, 0)),
            pl.BlockSpec((eb, de), lambda i: (i, 0)),
            pl.BlockSpec((de, d), lambda i: (0, 0)),
            pl.BlockSpec((1, d), lambda i: (0, 0)),
            pl.BlockSpec((d, d), lambda i: (0, 0)),
            pl.BlockSpec((1, d), lambda i: (0, 0)),
        ],
        out_specs=pl.BlockSpec((eb, d), lambda i: (i, 0)),
        out_shape=jax.ShapeDtypeStruct((e, d), F32),
    )(g, ea, we, b1, w2, b2)


def _update_mlp(h, aggr, wa, wb, b1, w2, b2, bn=1000):
    n, d = h.shape

    def body(h_ref, ag_ref, wa_ref, wb_ref, b1_ref, w2_ref, b2_ref, o_ref):
        hh = h_ref[...]
        z = (
            jnp.dot(hh, wa_ref[...], preferred_element_type=F32)
            + jnp.dot(ag_ref[...], wb_ref[...], preferred_element_type=F32)
            + b1_ref[...]
        )
        z = jnp.maximum(z, 0.0)
        u = jnp.dot(z, w2_ref[...], preferred_element_type=F32) + b2_ref[...]
        o_ref[...] = hh + jnp.maximum(u, 0.0)

    return pl.pallas_call(
        body,
        grid=(n // bn,),
        in_specs=[
            pl.BlockSpec((bn, d), lambda i: (i, 0)),
            pl.BlockSpec((bn, d), lambda i: (i, 0)),
            pl.BlockSpec((d, d), lambda i: (0, 0)),
            pl.BlockSpec((d, d), lambda i: (0, 0)),
            pl.BlockSpec((1, d), lambda i: (0, 0)),
            pl.BlockSpec((d, d), lambda i: (0, 0)),
            pl.BlockSpec((1, d), lambda i: (0, 0)),
        ],
        out_specs=pl.BlockSpec((bn, d), lambda i: (i, 0)),
        out_shape=jax.ShapeDtypeStruct((n, d), F32),
    )(h, aggr, wa, wb, b1, w2, b2)


def _pred_head(h, w, b, bn=1000):
    n, d = h.shape

    def body(h_ref, w_ref, b_ref, o_ref):
        o_ref[...] = (
            jnp.dot(h_ref[...], w_ref[...], preferred_element_type=F32)
            + b_ref[...]
        )

    return pl.pallas_call(
        body,
        grid=(n // bn,),
        in_specs=[
            pl.BlockSpec((bn, d), lambda i: (i, 0)),
            pl.BlockSpec((d, 1), lambda i: (0, 0)),
            pl.BlockSpec((1, 1), lambda i: (0, 0)),
        ],
        out_specs=pl.BlockSpec((bn, 1), lambda i: (i, 0)),
        out_shape=jax.ShapeDtypeStruct((n, 1), F32),
    )(h, w, b)


CHG = 40


@functools.cache
def _make_gather_add(n, e, d):
    nw = NC * NS
    ept = e // nw
    nch = ept // CHG
    mesh = plsc.VectorSubcoreMesh(core_axis_name="c", subcore_axis_name="s")

    @functools.partial(
        pl.kernel,
        out_type=jax.ShapeDtypeStruct((e, d), F32),
        mesh=mesh,
        scratch_types=[
            pltpu.VMEM((ept,), jnp.int32),
            pltpu.VMEM((ept,), jnp.int32),
            pltpu.VMEM((2, CHG, d), F32),
            pltpu.VMEM((2, CHG, d), F32),
            pltpu.VMEM((2, CHG, d), F32),
        ] + [pltpu.SemaphoreType.DMA] * 6,
    )
    def gather_add(a_hbm, b_hbm, dst_hbm, src_hbm, g_hbm,
                   dst_v, src_v, a_v, b_v, g_v,
                   sem_a0, sem_a1, sem_b0, sem_b1, sem_w0, sem_w1):
        sem_a = (sem_a0, sem_a1)
        sem_b = (sem_b0, sem_b1)
        sem_w = (sem_w0, sem_w1)
        wid = lax.axis_index("s") * NC + lax.axis_index("c")
        base = wid * ept
        pltpu.sync_copy(dst_hbm.at[pl.ds(base, ept)], dst_v)
        pltpu.sync_copy(src_hbm.at[pl.ds(base, ept)], src_v)

        def idx(ref, k):
            return ref.at[pl.ds(k * CHG, CHG)]

        for par in range(2):
            pltpu.async_copy(a_hbm.at[idx(dst_v, par)], a_v.at[par],
                             sem_a[par])
            pltpu.async_copy(b_hbm.at[idx(src_v, par)], b_v.at[par],
                             sem_b[par])

        def body(j2, carry):
            for par in range(2):
                k = 2 * j2 + par
                e0 = base + k * CHG
                pltpu.make_async_copy(
                    a_hbm.at[idx(dst_v, k)], a_v.at[par], sem_a[par]).wait()
                pltpu.make_async_copy(
                    b_hbm.at[idx(src_v, k)], b_v.at[par], sem_b[par]).wait()

                @pl.when(j2 >= 1)
                def _wait_writeback():
                    pltpu.make_async_copy(
                        g_v.at[par],
                        g_hbm.at[pl.ds(e0 - 2 * CHG, CHG)],
                        sem_w[par]).wait()

                def row(r, c2):
                    for j in range(d // LANES):
                        sl = pl.ds(j * LANES, LANES)
                        g_v[par, r, sl] = a_v[par, r, sl] + b_v[par, r, sl]
                    return c2

                lax.fori_loop(0, CHG, row, 0)
                pltpu.async_copy(g_v.at[par], g_hbm.at[pl.ds(e0, CHG)],
                                 sem_w[par])

                @pl.when(j2 < nch // 2 - 1)
                def _prefetch():
                    pltpu.async_copy(a_hbm.at[idx(dst_v, k + 2)],
                                     a_v.at[par], sem_a[par])
                    pltpu.async_copy(b_hbm.at[idx(src_v, k + 2)],
                                     b_v.at[par], sem_b[par])
            return carry

        lax.fori_loop(0, nch // 2, body, 0)
        for par in range(2):
            k = nch - 2 + par
            pltpu.make_async_copy(
                g_v.at[par], g_hbm.at[pl.ds(base + k * CHG, CHG)],
                sem_w[par]).wait()

    return gather_add


@functools.cache
def _make_scatter_add(n_pad, e, d):
    dh = d // NC
    eps = e // NS
    nch = eps // CH
    nps = n_pad // NS
    mesh = plsc.VectorSubcoreMesh(core_axis_name="c", subcore_axis_name="s")

    @functools.partial(
        pl.kernel,
        out_type=jax.ShapeDtypeStruct((n_pad, d), F32),
        mesh=mesh,
        scratch_types=[
            pltpu.VMEM_SHARED((n_pad, dh), F32),
            pltpu.VMEM((4, CH, dh), F32),
            pltpu.VMEM((CH,), jnp.int32),
            pltpu.VMEM((CH,), jnp.int32),
            pltpu.VMEM((CH,), jnp.int32),
            pltpu.VMEM((CH,), jnp.int32),
        ] + [pltpu.SemaphoreType.DMA] * 12,
    )
    def scatter_add(v_hbm, dst_hbm, z_hbm, aggr_hbm, acc_s, v_v,
                    i0, i1, i2, i3,
                    sv0, sv1, sv2, sv3, ss0, ss1, ss2, ss3,
                    si0, si1, si2, si3):
        idx = (i0, i1, i2, i3)
        sem_v = (sv0, sv1, sv2, sv3)
        sem_s = (ss0, ss1, ss2, ss3)
        sem_i = (si0, si1, si2, si3)
        cid = lax.axis_index("c")
        sid = lax.axis_index("s")
        r0 = sid * nps
        c0 = cid * dh
        pltpu.sync_copy(z_hbm.at[pl.ds(r0, nps)], acc_s.at[pl.ds(r0, nps)])
        plsc.subcore_barrier()
        base = sid * eps

        def load(k, par):
            pltpu.async_copy(dst_hbm.at[pl.ds(base + k * CH, CH)], idx[par],
                             sem_i[par])
            pltpu.async_copy(
                v_hbm.at[pl.ds(base + k * CH, CH), pl.ds(c0, dh)],
                v_v.at[par], sem_v[par])

        load(0, 0)

        def body(j4, carry):
            for par in range(4):
                k = 4 * j4 + par
                pltpu.make_async_copy(
                    dst_hbm.at[pl.ds(base + k * CH, CH)], idx[par],
                    sem_i[par]).wait()
                pltpu.make_async_copy(
                    v_hbm.at[pl.ds(base + k * CH, CH), pl.ds(c0, dh)],
                    v_v.at[par], sem_v[par]).wait()
                pltpu.async_copy(v_v.at[par], acc_s.at[idx[par]],
                                 sem_s[par], add=True)
                nxt = (par + 1) % 4

                def _load_next():
                    pltpu.make_async_copy(v_v.at[nxt], acc_s.at[idx[nxt]],
                                          sem_s[nxt]).wait()
                    load(k + 1, nxt)

                if par < 3:
                    @pl.when(j4 >= 1)
                    def _ln1():
                        _load_next()

                    @pl.when(j4 == 0)
                    def _ln2():
                        load(k + 1, nxt)
                else:
                    @pl.when(j4 < nch // 4 - 1)
                    def _ln3():
                        _load_next()
            return carry

        lax.fori_loop(0, nch // 4, body, 0)
        for par in range(4):
            pltpu.make_async_copy(v_v.at[par], acc_s.at[idx[par]],
                                  sem_s[par]).wait()
        plsc.subcore_barrier()
        pltpu.sync_copy(acc_s.at[pl.ds(r0, nps)],
                        aggr_hbm.at[pl.ds(r0, nps), pl.ds(c0, dh)])

    return scatter_add


def kernel(x, edge_index, edge_attr, W_in, b_in, Wm1, bm1, Wm2, bm2,
           Wu1, bu1, Wu2, bu2, W_pred, b_pred):
    n, _ = x.shape
    e = edge_index.shape[1]
    d = W_in.shape[1]
    nl = Wm1.shape[0]

    src = edge_index[0]
    dst = edge_index[1]
    n_pad = ((n + NS * 8 - 1) // (NS * 8)) * NS * 8
    zeros_half = jnp.zeros((n_pad, d // NC), F32)

    gather_add = _make_gather_add(n, e, d)
    scatter_add = _make_scatter_add(n_pad, e, d)

    h = _inproj(x, W_in, b_in.reshape(1, -1))
    for l in range(nl):
        a, b = _ab_proj(h, Wm1[l, :d], Wm1[l, d:2 * d])
        g = gather_add(a, b, dst, src)
        v = _edge_mlp(g, edge_attr, Wm1[l, 2 * d:], bm1[l].reshape(1, -1),
                      Wm2[l], bm2[l].reshape(1, -1))
        aggr = scatter_add(v, dst, zeros_half)[:n]
        h = _update_mlp(h, aggr, Wu1[l, :d], Wu1[l, d:], bu1[l].reshape(1, -1),
                        Wu2[l], bu2[l].reshape(1, -1))
    return _pred_head(h, W_pred, b_pred.reshape(1, -1))


# speedup vs baseline: 2.1416x; 1.0390x over previous
import functools

import jax
import jax.numpy as jnp
from jax import lax
from jax.experimental import pallas as pl
from jax.experimental.pallas import tpu as pltpu
from jax.experimental.pallas import tpu_sc as plsc

NC = 2
NS = 16
LANES = 16
CH = 128
CHG = 64

F32 = jnp.float32
BF16 = jnp.bfloat16


def _inproj(x, w, b, bn=1000):
    n, din = x.shape
    d = w.shape[1]

    def body(x_ref, w_ref, b_ref, o_ref):
        o_ref[...] = (
            jnp.dot(x_ref[...], w_ref[...], preferred_element_type=F32)
            + b_ref[...]
        )

    return pl.pallas_call(
        body,
        grid=(n // bn,),
        in_specs=[
            pl.BlockSpec((bn, din), lambda i: (i, 0)),
            pl.BlockSpec((din, d), lambda i: (0, 0)),
            pl.BlockSpec((1, d), lambda i: (0, 0)),
        ],
        out_specs=pl.BlockSpec((bn, d), lambda i: (i, 0)),
        out_shape=jax.ShapeDtypeStruct((n, d), F32),
    )(x, w, b)


def _ab_proj(h, wa, wb, bn=1000):
    n, d = h.shape

    dh = d // 2

    def pack(m):
        lo = lax.bitcast_convert_type(
            m[:, :dh].astype(BF16), jnp.uint16).astype(jnp.int32)
        hi = lax.bitcast_convert_type(
            m[:, dh:].astype(BF16), jnp.uint16).astype(jnp.int32)
        return lo | (hi << 16)

    def body(h_ref, wa_ref, wb_ref, a_ref, b_ref):
        hh = h_ref[...]
        a_ref[...] = pack(jnp.dot(hh, wa_ref[...],
                                  preferred_element_type=F32))
        b_ref[...] = pack(jnp.dot(hh, wb_ref[...],
                                  preferred_element_type=F32))

    return pl.pallas_call(
        body,
        grid=(n // bn,),
        in_specs=[
            pl.BlockSpec((bn, d), lambda i: (i, 0)),
            pl.BlockSpec((d, d), lambda i: (0, 0)),
            pl.BlockSpec((d, d), lambda i: (0, 0)),
        ],
        out_specs=[
            pl.BlockSpec((bn, d // 2), lambda i: (i, 0)),
            pl.BlockSpec((bn, d // 2), lambda i: (i, 0)),
        ],
        out_shape=[
            jax.ShapeDtypeStruct((n, d // 2), jnp.int32),
            jax.ShapeDtypeStruct((n, d // 2), jnp.int32),
        ],
    )(h, wa, wb)


def _edge_mlp(ag, bg, ea, we, b1, w2, b2, eb=1280):
    e = ag.shape[0]
    d = ag.shape[1] * 2
    de = ea.shape[1]
    dh = d // NC

    def unpack_lo(w):
        return lax.bitcast_convert_type(w << 16, F32)

    def unpack_hi(w):
        return lax.bitcast_convert_type(w & jnp.int32(-65536), F32)

    def body(ag_ref, bg_ref, ea_ref, we_ref, b1_ref, w2_ref, b2_ref, o_ref):
        a32 = ag_ref[...]
        b32 = bg_ref[...]
        lo = unpack_lo(a32) + unpack_lo(b32)
        hi = unpack_hi(a32) + unpack_hi(b32)
        t = (
            jnp.concatenate([lo, hi], axis=1)
            + jnp.dot(ea_ref[...], we_ref[...], preferred_element_type=F32)
            + b1_ref[...]
        )
        u = jnp.maximum(t, 0.0)
        v = jnp.dot(u.astype(BF16), w2_ref[...],
                    preferred_element_type=F32) + b2_ref[...]
        v = jnp.maximum(v, 0.0)
        o_ref[0] = v[:, :dh]
        o_ref[1] = v[:, dh:]

    return pl.pallas_call(
        body,
        grid=(e // eb,),
        in_specs=[
            pl.BlockSpec((eb, d // 2), lambda i: (i, 0)),
            pl.BlockSpec((eb, d // 2), lambda i: (i, 0)),
            pl.BlockSpec((eb, de), lambda i: (i, 0)),
            pl.BlockSpec((de, d), lambda i: (0, 0)),
            pl.BlockSpec((1, d), lambda i: (0, 0)),
            pl.BlockSpec((d, d), lambda i: (0, 0)),
            pl.BlockSpec((1, d), lambda i: (0, 0)),
        ],
        out_specs=pl.BlockSpec((2, eb, dh), lambda i: (0, i, 0)),
        out_shape=jax.ShapeDtypeStruct((2, e, dh), F32),
    )(ag, bg, ea, we, b1, w2, b2)


def _update_mlp(h, aggr, wa, wb, b1, w2, b2, bn=1000):
    n, d = h.shape

    def body(h_ref, ag_ref, wa_ref, wb_ref, b1_ref, w2_ref, b2_ref, o_ref):
        hh = h_ref[...]
        z = (
            jnp.dot(hh, wa_ref[...], preferred_element_type=F32)
            + jnp.dot(ag_ref[...], wb_ref[...], preferred_element_type=F32)
            + b1_ref[...]
        )
        z = jnp.maximum(z, 0.0)
        u = jnp.dot(z, w2_ref[...], preferred_element_type=F32) + b2_ref[...]
        o_ref[...] = hh + jnp.maximum(u, 0.0)

    return pl.pallas_call(
        body,
        grid=(n // bn,),
        in_specs=[
            pl.BlockSpec((bn, d), lambda i: (i, 0)),
            pl.BlockSpec((bn, d), lambda i: (i, 0)),
            pl.BlockSpec((d, d), lambda i: (0, 0)),
            pl.BlockSpec((d, d), lambda i: (0, 0)),
            pl.BlockSpec((1, d), lambda i: (0, 0)),
            pl.BlockSpec((d, d), lambda i: (0, 0)),
            pl.BlockSpec((1, d), lambda i: (0, 0)),
        ],
        out_specs=pl.BlockSpec((bn, d), lambda i: (i, 0)),
        out_shape=jax.ShapeDtypeStruct((n, d), F32),
    )(h, aggr, wa, wb, b1, w2, b2)


def _pred_head(h, w, b, bn=1000):
    n, d = h.shape

    def body(h_ref, w_ref, b_ref, o_ref):
        o_ref[...] = (
            jnp.dot(h_ref[...], w_ref[...], preferred_element_type=F32)
            + b_ref[...]
        )

    return pl.pallas_call(
        body,
        grid=(n // bn,),
        in_specs=[
            pl.BlockSpec((bn, d), lambda i: (i, 0)),
            pl.BlockSpec((d, 1), lambda i: (0, 0)),
            pl.BlockSpec((1, 1), lambda i: (0, 0)),
        ],
        out_specs=pl.BlockSpec((bn, 1), lambda i: (i, 0)),
        out_shape=jax.ShapeDtypeStruct((n, 1), F32),
    )(h, w, b)


@functools.cache
def _make_gather_add(n, e, d):
    nw = NC * NS
    ept = e // nw
    nch = ept // CHG
    dw = d // 2
    mesh = plsc.VectorSubcoreMesh(core_axis_name="c", subcore_axis_name="s")

    @functools.partial(
        pl.kernel,
        out_type=(jax.ShapeDtypeStruct((e, dw), jnp.int32),
                  jax.ShapeDtypeStruct((e, dw), jnp.int32)),
        mesh=mesh,
        scratch_types=[
            pltpu.VMEM((ept,), jnp.int32),
            pltpu.VMEM((ept,), jnp.int32),
            pltpu.VMEM((4, CHG, dw), jnp.int32),
            pltpu.VMEM((4, CHG, dw), jnp.int32),
        ] + [pltpu.SemaphoreType.DMA] * 16,
    )
    def gather2(a_hbm, b_hbm, dst_hbm, src_hbm, ag_hbm, bg_hbm,
                dst_v, src_v, a_v, b_v,
                sa0, sa1, sa2, sa3, sb0, sb1, sb2, sb3,
                wa0, wa1, wa2, wa3, wb0, wb1, wb2, wb3):
        sem_a = (sa0, sa1, sa2, sa3)
        sem_b = (sb0, sb1, sb2, sb3)
        sem_wa = (wa0, wa1, wa2, wa3)
        sem_wb = (wb0, wb1, wb2, wb3)
        wid = lax.axis_index("s") * NC + lax.axis_index("c")
        base = wid * ept
        pltpu.sync_copy(dst_hbm.at[pl.ds(base, ept)], dst_v)
        pltpu.sync_copy(src_hbm.at[pl.ds(base, ept)], src_v)

        def idx(ref, k):
            return ref.at[pl.ds(k * CHG, CHG)]

        def gath(k, par):
            pltpu.async_copy(a_hbm.at[idx(dst_v, k)], a_v.at[par],
                             sem_a[par])
            pltpu.async_copy(b_hbm.at[idx(src_v, k)], b_v.at[par],
                             sem_b[par])

        gath(0, 0)

        def body(j4, carry):
            for par in range(4):
                k = 4 * j4 + par
                e0 = base + k * CHG
                pltpu.make_async_copy(
                    a_hbm.at[idx(dst_v, k)], a_v.at[par], sem_a[par]).wait()
                pltpu.make_async_copy(
                    b_hbm.at[idx(src_v, k)], b_v.at[par], sem_b[par]).wait()
                pltpu.async_copy(a_v.at[par], ag_hbm.at[pl.ds(e0, CHG)],
                                 sem_wa[par])
                pltpu.async_copy(b_v.at[par], bg_hbm.at[pl.ds(e0, CHG)],
                                 sem_wb[par])
                nxt = (par + 1) % 4

                def _gath_next():
                    pltpu.make_async_copy(
                        a_v.at[nxt], ag_hbm.at[pl.ds(e0, CHG)],
                        sem_wa[nxt]).wait()
                    pltpu.make_async_copy(
                        b_v.at[nxt], bg_hbm.at[pl.ds(e0, CHG)],
                        sem_wb[nxt]).wait()
                    gath(k + 1, nxt)

                if par < 3:
                    @pl.when(j4 >= 1)
                    def _g1():
                        _gath_next()

                    @pl.when(j4 == 0)
                    def _g2():
                        gath(k + 1, nxt)
                else:
                    @pl.when(j4 < nch // 4 - 1)
                    def _g3():
                        _gath_next()
            return carry

        lax.fori_loop(0, nch // 4, body, 0)
        for par in range(4):
            k = nch - 4 + par
            e0 = base + k * CHG
            pltpu.make_async_copy(a_v.at[par], ag_hbm.at[pl.ds(e0, CHG)],
                                  sem_wa[par]).wait()
            pltpu.make_async_copy(b_v.at[par], bg_hbm.at[pl.ds(e0, CHG)],
                                  sem_wb[par]).wait()

    return gather2


@functools.cache
def _make_scatter_add(n_pad, e, d):
    dh = d // NC
    eps = e // NS
    nch = eps // CH
    nps = n_pad // NS
    mesh = plsc.VectorSubcoreMesh(core_axis_name="c", subcore_axis_name="s")

    @functools.partial(
        pl.kernel,
        out_type=jax.ShapeDtypeStruct((n_pad, d), F32),
        mesh=mesh,
        scratch_types=[
            pltpu.VMEM_SHARED((n_pad, dh), F32),
            pltpu.VMEM((2, CH, dh), F32),
            pltpu.VMEM((CH,), jnp.int32),
            pltpu.VMEM((CH,), jnp.int32),
        ] + [pltpu.SemaphoreType.DMA] * 6,
    )
    def scatter_add(v_hbm, dst_hbm, z_hbm, aggr_hbm, acc_s, v_v,
                    i0, i1, sv0, sv1, ss0, ss1, si0, si1):
        idx = (i0, i1)
        sem_v = (sv0, sv1)
        sem_s = (ss0, ss1)
        sem_i = (si0, si1)
        cid = lax.axis_index("c")
        sid = lax.axis_index("s")
        r0 = sid * nps
        c0 = cid * dh
        pltpu.sync_copy(z_hbm.at[pl.ds(r0, nps)], acc_s.at[pl.ds(r0, nps)])
        plsc.subcore_barrier()
        base = sid * eps

        def load(k, par):
            pltpu.async_copy(dst_hbm.at[pl.ds(base + k * CH, CH)], idx[par],
                             sem_i[par])
            pltpu.async_copy(v_hbm.at[cid, pl.ds(base + k * CH, CH)],
                             v_v.at[par], sem_v[par])

        load(0, 0)

        def body(j2, carry):
            for par in range(2):
                k = 2 * j2 + par
                pltpu.make_async_copy(
                    dst_hbm.at[pl.ds(base + k * CH, CH)], idx[par],
                    sem_i[par]).wait()
                pltpu.make_async_copy(
                    v_hbm.at[cid, pl.ds(base + k * CH, CH)],
                    v_v.at[par], sem_v[par]).wait()
                pltpu.async_copy(v_v.at[par], acc_s.at[idx[par]],
                                 sem_s[par], add=True)
                nxt = (par + 1) % 2

                def _load_next():
                    pltpu.make_async_copy(v_v.at[nxt], acc_s.at[idx[nxt]],
                                          sem_s[nxt]).wait()
                    load(k + 1, nxt)

                if par == 0:
                    @pl.when(j2 >= 1)
                    def _ln1():
                        _load_next()

                    @pl.when(j2 == 0)
                    def _ln2():
                        load(k + 1, nxt)
                else:
                    @pl.when(j2 < nch // 2 - 1)
                    def _ln3():
                        _load_next()
            return carry

        lax.fori_loop(0, nch // 2, body, 0)
        for par in range(2):
            pltpu.make_async_copy(v_v.at[par], acc_s.at[idx[par]],
                                  sem_s[par]).wait()
        plsc.subcore_barrier()
        pltpu.sync_copy(acc_s.at[pl.ds(r0, nps)],
                        aggr_hbm.at[pl.ds(r0, nps), pl.ds(c0, dh)])

    return scatter_add


def kernel(x, edge_index, edge_attr, W_in, b_in, Wm1, bm1, Wm2, bm2,
           Wu1, bu1, Wu2, bu2, W_pred, b_pred):
    n, _ = x.shape
    e = edge_index.shape[1]
    d = W_in.shape[1]
    nl = Wm1.shape[0]

    src = edge_index[0]
    dst = edge_index[1]
    n_pad = ((n + NS * 8 - 1) // (NS * 8)) * NS * 8
    quant = NC * NS * CHG * 4
    e_pad = ((e + quant - 1) // quant) * quant
    pad = e_pad - e
    dst_g = jnp.pad(dst, (0, pad))
    src_g = jnp.pad(src, (0, pad))
    dst_s = jnp.pad(dst, (0, pad), constant_values=n_pad - 1)
    ea_p = jnp.pad(edge_attr, ((0, pad), (0, 0)))
    zeros_half = jnp.zeros((n_pad, d // NC), F32)

    gather_add = _make_gather_add(n, e_pad, d)
    scatter_add = _make_scatter_add(n_pad, e_pad, d)

    h = _inproj(x, W_in, b_in.reshape(1, -1))
    for l in range(nl):
        a, b = _ab_proj(h, Wm1[l, :d], Wm1[l, d:2 * d])
        ag, bg = gather_add(a, b, dst_g, src_g)
        v = _edge_mlp(ag, bg, ea_p, Wm1[l, 2 * d:], bm1[l].reshape(1, -1),
                      Wm2[l].astype(BF16), bm2[l].reshape(1, -1))
        aggr = scatter_add(v, dst_s, zeros_half)[:n]
        h = _update_mlp(h, aggr, Wu1[l, :d], Wu1[l, d:], bu1[l].reshape(1, -1),
                        Wu2[l], bu2[l].reshape(1, -1))
    return _pred_head(h, W_pred, b_pred.reshape(1, -1))


# speedup vs baseline: 2.2289x; 1.0408x over previous
import functools
import math

import jax
import jax.numpy as jnp
from jax import lax
from jax.experimental import pallas as pl
from jax.experimental.pallas import tpu as pltpu
from jax.experimental.pallas import tpu_sc as plsc

NC = 2
NS = 16
LANES = 16
CH = 128
CHG = 64

F32 = jnp.float32
BF16 = jnp.bfloat16


def _inproj(x, w, b, bn=1000):
    n, din = x.shape
    d = w.shape[1]

    def body(x_ref, w_ref, b_ref, o_ref):
        o_ref[...] = (
            jnp.dot(x_ref[...], w_ref[...], preferred_element_type=F32)
            + b_ref[...]
        )

    return pl.pallas_call(
        body,
        grid=(n // bn,),
        in_specs=[
            pl.BlockSpec((bn, din), lambda i: (i, 0)),
            pl.BlockSpec((din, d), lambda i: (0, 0)),
            pl.BlockSpec((1, d), lambda i: (0, 0)),
        ],
        out_specs=pl.BlockSpec((bn, d), lambda i: (i, 0)),
        out_shape=jax.ShapeDtypeStruct((n, d), F32),
    )(x, w, b)


def _ab_proj(h, wa, wb, bn=1000):
    n, d = h.shape

    dh = d // 2

    def pack(m):
        lo = lax.bitcast_convert_type(
            m[:, :dh].astype(BF16), jnp.uint16).astype(jnp.int32)
        hi = lax.bitcast_convert_type(
            m[:, dh:].astype(BF16), jnp.uint16).astype(jnp.int32)
        return lo | (hi << 16)

    def body(h_ref, wa_ref, wb_ref, o_ref):
        hh = h_ref[...]
        o_ref[0] = pack(jnp.dot(hh, wa_ref[...], preferred_element_type=F32))
        o_ref[1] = pack(jnp.dot(hh, wb_ref[...], preferred_element_type=F32))

    return pl.pallas_call(
        body,
        grid=(n // bn,),
        in_specs=[
            pl.BlockSpec((bn, d), lambda i: (i, 0)),
            pl.BlockSpec((d, d), lambda i: (0, 0)),
            pl.BlockSpec((d, d), lambda i: (0, 0)),
        ],
        out_specs=pl.BlockSpec((2, bn, d // 2), lambda i: (0, i, 0)),
        out_shape=jax.ShapeDtypeStruct((2, n, d // 2), jnp.int32),
    )(h, wa, wb)


def _edge_mlp(ag, bg, ea, we, b1, w2, b2, eb=1280):
    e = ag.shape[0]
    d = ag.shape[1] * 2
    de = ea.shape[1]
    dh = d // NC

    def unpack_lo(w):
        return lax.bitcast_convert_type(w << 16, F32)

    def unpack_hi(w):
        return lax.bitcast_convert_type(w & jnp.int32(-65536), F32)

    def body(ag_ref, bg_ref, ea_ref, we_ref, b1_ref, w2_ref, b2_ref, o_ref):
        a32 = ag_ref[...]
        b32 = bg_ref[...]
        lo = unpack_lo(a32) + unpack_lo(b32)
        hi = unpack_hi(a32) + unpack_hi(b32)
        t = (
            jnp.concatenate([lo, hi], axis=1)
            + jnp.dot(ea_ref[...], we_ref[...], preferred_element_type=F32)
            + b1_ref[...]
        )
        u = jnp.maximum(t, 0.0)
        v = jnp.dot(u.astype(BF16), w2_ref[...],
                    preferred_element_type=F32) + b2_ref[...]
        v = jnp.maximum(v, 0.0)
        o_ref[0] = v[:, :dh]
        o_ref[1] = v[:, dh:]

    return pl.pallas_call(
        body,
        grid=(e // eb,),
        in_specs=[
            pl.BlockSpec((eb, d // 2), lambda i: (i, 0)),
            pl.BlockSpec((eb, d // 2), lambda i: (i, 0)),
            pl.BlockSpec((eb, de), lambda i: (i, 0)),
            pl.BlockSpec((de, d), lambda i: (0, 0)),
            pl.BlockSpec((1, d), lambda i: (0, 0)),
            pl.BlockSpec((d, d), lambda i: (0, 0)),
            pl.BlockSpec((1, d), lambda i: (0, 0)),
        ],
        out_specs=pl.BlockSpec((2, eb, dh), lambda i: (0, i, 0)),
        out_shape=jax.ShapeDtypeStruct((2, e, dh), F32),
    )(ag, bg, ea, we, b1, w2, b2)


def _update_mlp(h, aggr, wa, wb, b1, w2, b2, bn=1000):
    n, d = h.shape

    def body(h_ref, ag_ref, wa_ref, wb_ref, b1_ref, w2_ref, b2_ref, o_ref):
        hh = h_ref[...]
        z = (
            jnp.dot(hh, wa_ref[...], preferred_element_type=F32)
            + jnp.dot(ag_ref[...], wb_ref[...], preferred_element_type=F32)
            + b1_ref[...]
        )
        z = jnp.maximum(z, 0.0)
        u = jnp.dot(z, w2_ref[...], preferred_element_type=F32) + b2_ref[...]
        o_ref[...] = hh + jnp.maximum(u, 0.0)

    return pl.pallas_call(
        body,
        grid=(n // bn,),
        in_specs=[
            pl.BlockSpec((bn, d), lambda i: (i, 0)),
            pl.BlockSpec((bn, d), lambda i: (i, 0)),
            pl.BlockSpec((d, d), lambda i: (0, 0)),
            pl.BlockSpec((d, d), lambda i: (0, 0)),
            pl.BlockSpec((1, d), lambda i: (0, 0)),
            pl.BlockSpec((d, d), lambda i: (0, 0)),
            pl.BlockSpec((1, d), lambda i: (0, 0)),
        ],
        out_specs=pl.BlockSpec((bn, d), lambda i: (i, 0)),
        out_shape=jax.ShapeDtypeStruct((n, d), F32),
    )(h, aggr, wa, wb, b1, w2, b2)


def _pred_head(h, w, b, bn=1000):
    n, d = h.shape

    def body(h_ref, w_ref, b_ref, o_ref):
        o_ref[...] = (
            jnp.dot(h_ref[...], w_ref[...], preferred_element_type=F32)
            + b_ref[...]
        )

    return pl.pallas_call(
        body,
        grid=(n // bn,),
        in_specs=[
            pl.BlockSpec((bn, d), lambda i: (i, 0)),
            pl.BlockSpec((d, 1), lambda i: (0, 0)),
            pl.BlockSpec((1, 1), lambda i: (0, 0)),
        ],
        out_specs=pl.BlockSpec((bn, 1), lambda i: (i, 0)),
        out_shape=jax.ShapeDtypeStruct((n, 1), F32),
    )(h, w, b)


@functools.cache
def _make_gather_add(n, e, d):
    nw = NC * NS
    ept = e // nw
    nch = ept // CHG
    dw = d // 2
    mesh = plsc.VectorSubcoreMesh(core_axis_name="c", subcore_axis_name="s")

    @functools.partial(
        pl.kernel,
        out_type=(jax.ShapeDtypeStruct((e, dw), jnp.int32),
                  jax.ShapeDtypeStruct((e, dw), jnp.int32)),
        mesh=mesh,
        scratch_types=[
            pltpu.VMEM((2 * ept,), jnp.int32),
            pltpu.VMEM((4, 2 * CHG, dw), jnp.int32),
        ] + [pltpu.SemaphoreType.DMA] * 12,
    )
    def gather2(t_hbm, cidx_hbm, ag_hbm, bg_hbm, cidx_v, ab_v, *sems):
        sem_g = sems[0:4]
        sem_wa = sems[4:8]
        sem_wb = sems[8:12]
        wid = lax.axis_index("s") * NC + lax.axis_index("c")
        base = wid * ept
        pltpu.sync_copy(cidx_hbm.at[pl.ds(2 * base, 2 * ept)], cidx_v)

        def gath(k, par):
            pltpu.async_copy(
                t_hbm.at[cidx_v.at[pl.ds(k * 2 * CHG, 2 * CHG)]],
                ab_v.at[par], sem_g[par])

        def wait_wb(k, par):
            e0 = base + k * CHG
            pltpu.make_async_copy(
                ab_v.at[par, pl.ds(0, CHG)], ag_hbm.at[pl.ds(e0, CHG)],
                sem_wa[par]).wait()
            pltpu.make_async_copy(
                ab_v.at[par, pl.ds(CHG, CHG)], bg_hbm.at[pl.ds(e0, CHG)],
                sem_wb[par]).wait()

        gath(0, 0)
        gath(1, 1)
        nj = nch // 4

        def body(j4, carry):
            for par in range(4):
                k = 4 * j4 + par
                e0 = base + k * CHG
                pltpu.make_async_copy(
                    t_hbm.at[cidx_v.at[pl.ds(k * 2 * CHG, 2 * CHG)]],
                    ab_v.at[par], sem_g[par]).wait()
                pltpu.async_copy(
                    ab_v.at[par, pl.ds(0, CHG)], ag_hbm.at[pl.ds(e0, CHG)],
                    sem_wa[par])
                pltpu.async_copy(
                    ab_v.at[par, pl.ds(CHG, CHG)], bg_hbm.at[pl.ds(e0, CHG)],
                    sem_wb[par])
                nxt = (par + 2) % 4

                def _gath_next():
                    wait_wb(k - 2, nxt)
                    gath(k + 2, nxt)

                if par < 2:
                    @pl.when(j4 >= 1)
                    def _g1():
                        _gath_next()

                    @pl.when(j4 == 0)
                    def _g2():
                        gath(k + 2, nxt)
                else:
                    @pl.when(j4 < nj - 1)
                    def _g3():
                        _gath_next()
            return carry

        lax.fori_loop(0, nj, body, 0)
        for par in range(4):
            wait_wb(nch - 4 + par, par)

    return gather2


@functools.cache
def _make_scatter_add(n_pad, e, d):
    dh = d // NC
    eps = e // NS
    nch = eps // CH
    nps = n_pad // NS
    mesh = plsc.VectorSubcoreMesh(core_axis_name="c", subcore_axis_name="s")

    @functools.partial(
        pl.kernel,
        out_type=jax.ShapeDtypeStruct((n_pad, d), F32),
        mesh=mesh,
        scratch_types=[
            pltpu.VMEM_SHARED((n_pad, dh), F32),
            pltpu.VMEM((2, CH, dh), F32),
            pltpu.VMEM((CH,), jnp.int32),
            pltpu.VMEM((CH,), jnp.int32),
        ] + [pltpu.SemaphoreType.DMA] * 6,
    )
    def scatter_add(v_hbm, dst_hbm, z_hbm, aggr_hbm, acc_s, v_v,
                    i0, i1, *sems):
        idx = (i0, i1)
        sem_i = sems[0:2]
        sem_v = sems[2:4]
        sem_s = sems[4:6]
        cid = lax.axis_index("c")
        sid = lax.axis_index("s")
        r0 = sid * nps
        c0 = cid * dh
        pltpu.sync_copy(z_hbm.at[pl.ds(r0, nps)], acc_s.at[pl.ds(r0, nps)])
        plsc.subcore_barrier()
        base = sid * eps

        def load(k, par):
            pltpu.async_copy(dst_hbm.at[pl.ds(base + k * CH, CH)], idx[par],
                             sem_i[par])
            pltpu.async_copy(v_hbm.at[cid, pl.ds(base + k * CH, CH)],
                             v_v.at[par], sem_v[par])

        load(0, 0)

        def body(j2, carry):
            for par in range(2):
                k = 2 * j2 + par
                pltpu.make_async_copy(
                    dst_hbm.at[pl.ds(base + k * CH, CH)], idx[par],
                    sem_i[par]).wait()
                pltpu.make_async_copy(
                    v_hbm.at[cid, pl.ds(base + k * CH, CH)],
                    v_v.at[par], sem_v[par]).wait()
                pltpu.async_copy(v_v.at[par], acc_s.at[idx[par]],
                                 sem_s[par], add=True)
                nxt = (par + 1) % 2

                def _load_next():
                    pltpu.make_async_copy(v_v.at[nxt], acc_s.at[idx[nxt]],
                                          sem_s[nxt]).wait()
                    load(k + 1, nxt)

                if par == 0:
                    @pl.when(j2 >= 1)
                    def _ln1():
                        _load_next()

                    @pl.when(j2 == 0)
                    def _ln2():
                        load(k + 1, nxt)
                else:
                    @pl.when(j2 < nch // 2 - 1)
                    def _ln3():
                        _load_next()
            return carry

        lax.fori_loop(0, nch // 2, body, 0)
        for par in range(2):
            pltpu.make_async_copy(v_v.at[par], acc_s.at[idx[par]],
                                  sem_s[par]).wait()
        plsc.subcore_barrier()
        pltpu.sync_copy(acc_s.at[pl.ds(r0, nps)],
                        aggr_hbm.at[pl.ds(r0, nps), pl.ds(c0, dh)])

    return scatter_add


def kernel(x, edge_index, edge_attr, W_in, b_in, Wm1, bm1, Wm2, bm2,
           Wu1, bu1, Wu2, bu2, W_pred, b_pred):
    n, _ = x.shape
    e = edge_index.shape[1]
    d = W_in.shape[1]
    nl = Wm1.shape[0]

    src = edge_index[0]
    dst = edge_index[1]
    n_pad = ((n + NS * 8 - 1) // (NS * 8)) * NS * 8
    quant = math.lcm(NC * NS * CHG * 4, NS * CH * 2)
    e_pad = ((e + quant - 1) // quant) * quant
    pad = e_pad - e
    dst_g = jnp.pad(dst, (0, pad))
    src_g = jnp.pad(src, (0, pad))
    dst_s = jnp.pad(dst, (0, pad), constant_values=n_pad - 1)
    ea_p = jnp.pad(edge_attr, ((0, pad), (0, 0)))
    cidx = jnp.concatenate(
        [dst_g.reshape(-1, CHG), src_g.reshape(-1, CHG) + n],
        axis=1).reshape(-1)
    zeros_half = jnp.zeros((n_pad, d // NC), F32)

    gather_add = _make_gather_add(n, e_pad, d)
    scatter_add = _make_scatter_add(n_pad, e_pad, d)

    h = _inproj(x, W_in, b_in.reshape(1, -1))
    for l in range(nl):
        ab = _ab_proj(h, Wm1[l, :d], Wm1[l, d:2 * d])
        ag, bg = gather_add(ab.reshape(2 * n, d // 2), cidx)
        v = _edge_mlp(ag, bg, ea_p, Wm1[l, 2 * d:], bm1[l].reshape(1, -1),
                      Wm2[l].astype(BF16), bm2[l].reshape(1, -1))
        aggr = scatter_add(v, dst_s, zeros_half)[:n]
        h = _update_mlp(h, aggr, Wu1[l, :d], Wu1[l, d:], bu1[l].reshape(1, -1),
                        Wu2[l], bu2[l].reshape(1, -1))
    return _pred_head(h, W_pred, b_pred.reshape(1, -1))
